# baseline (speedup 1.0000x reference)
import functools

import jax
import jax.numpy as jnp
from jax import lax
from jax.experimental import pallas as pl
from jax.experimental.pallas import tpu as pltpu
from jax.experimental.pallas import tpu_sc as plsc

N_NEURONS = 8192
D_STATE = 256
B = 4096

NC = 2
NS = 16
NW = NC * NS
B_PER_W = B // NW
LANES = 16
G = 4
NGROUPS = B_PER_W // G
SIG_CHUNK = 32


def _body(idx_hbm, idx4_hbm, sig_hbm, conn_hbm, sigs_out, conns_out,
          idx_all, idx4_mine, sig_buf,
          row0, row1, out0, out1,
          sem_in0, sem_in1, sem_out0, sem_out1, sem_sig):
    wid = lax.axis_index("s") * NC + lax.axis_index("c")
    base = wid * B_PER_W
    gbase = wid * NGROUPS

    pltpu.sync_copy(idx_hbm, idx_all)
    pltpu.sync_copy(idx4_hbm.at[pl.ds(gbase, NGROUPS)], idx4_mine)

    def sig_step(c, _):
        pltpu.async_copy(
            sig_hbm.at[idx_all.at[pl.ds(base + c * SIG_CHUNK, SIG_CHUNK)]],
            sig_buf, sem_sig).wait()
        pltpu.sync_copy(sig_buf,
                        sigs_out.at[pl.ds(base + c * SIG_CHUNK, SIG_CHUNK)])
        return ()

    lax.fori_loop(0, B_PER_W // SIG_CHUNK, sig_step, ())

    rows = (row0, row1)
    outs = (out0, out1)
    sems_in = (sem_in0, sem_in1)
    sems_out = (sem_out0, sem_out1)

    def start_in(g, b):
        pltpu.async_copy(conn_hbm.at[idx4_mine.at[g]], rows[b], sems_in[b])

    def wait_in(b):
        pltpu.make_async_copy(conn_hbm.at[idx4_mine.at[0]], rows[b],
                              sems_in[b]).wait()

    def start_out(g, b):
        pltpu.async_copy(outs[b],
                         conns_out.at[pl.ds((base + g * G) * B, G * B)],
                         sems_out[b])

    def wait_out(b):
        pltpu.make_async_copy(outs[b], conns_out.at[pl.ds(0, G * B)],
                              sems_out[b]).wait()

    def compute(b):
        rbuf = rows[b]
        obuf = outs[b]

        @plsc.parallel_loop(0, B // LANES, 1, unroll=8)
        def col_step(k):
            cols = idx_all[pl.ds(k * LANES, LANES)]
            for j in range(G):
                rj = jnp.full((LANES,), j, jnp.int32)
                vals = plsc.load_gather(rbuf, [rj, cols])
                obuf[pl.ds(j * B + k * LANES, LANES)] = vals

    start_in(0, 0)
    start_in(1, 1)

    def group_pair(h, _):
        for b in range(2):
            g = 2 * h + b
            wait_in(b)

            @pl.when(h > 0)
            def _():
                wait_out(b)

            compute(b)
            start_out(g, b)

            @pl.when(g + 2 < NGROUPS)
            def _():
                start_in(g + 2, b)
        return ()

    lax.fori_loop(0, NGROUPS // 2, group_pair, ())
    wait_out(0)
    wait_out(1)


@jax.jit
def _run(indices, idx4, neuron_signatures, connection_strength):
    mesh = plsc.VectorSubcoreMesh(core_axis_name="c", subcore_axis_name="s",
                                  num_cores=NC, num_subcores=NS)
    kern = functools.partial(
        pl.kernel,
        out_type=[
            jax.ShapeDtypeStruct((B, D_STATE), jnp.float32),
            jax.ShapeDtypeStruct((B * B,), jnp.float32),
        ],
        mesh=mesh,
        compiler_params=pltpu.CompilerParams(needs_layout_passes=False),
        scratch_types=[
            pltpu.VMEM((B,), jnp.int32),
            pltpu.VMEM((NGROUPS, G), jnp.int32),
            pltpu.VMEM((SIG_CHUNK, D_STATE), jnp.float32),
            pltpu.VMEM((G, N_NEURONS), jnp.float32),
            pltpu.VMEM((G, N_NEURONS), jnp.float32),
            pltpu.VMEM((G * B,), jnp.float32),
            pltpu.VMEM((G * B,), jnp.float32),
            pltpu.SemaphoreType.DMA,
            pltpu.SemaphoreType.DMA,
            pltpu.SemaphoreType.DMA,
            pltpu.SemaphoreType.DMA,
            pltpu.SemaphoreType.DMA,
        ],
    )(_body)
    sigs, conns = kern(indices, idx4, neuron_signatures, connection_strength)
    return sigs, conns.reshape(B, B)


def kernel(indices, neuron_signatures, connection_strength):
    idx = indices.astype(jnp.int32)
    return _run(idx, idx.reshape(B // G, G), neuron_signatures,
                connection_strength)


# speedup vs baseline: 1.6349x; 1.6349x over previous
import functools

import jax
import jax.numpy as jnp
from jax import lax
from jax.experimental import pallas as pl
from jax.experimental.pallas import tpu as pltpu
from jax.experimental.pallas import tpu_sc as plsc

N_NEURONS = 8192
D_STATE = 256
B = 4096

NC = 2
NS = 16
NW = NC * NS
B_PER_W = B // NW
LANES = 16
G = 4
NGROUPS = B_PER_W // G
SIG_CHUNK = 32


def _body(idx_hbm, idx4_hbm, sig_hbm, conn_hbm, sigs_out, conns_out,
          idx_all, idx4_mine, sig_buf,
          row0, row1, out0, out1,
          sem_in0, sem_in1, sem_out0, sem_out1, sem_sig):
    wid = lax.axis_index("s") * NC + lax.axis_index("c")
    base = wid * B_PER_W
    gbase = wid * NGROUPS

    pltpu.sync_copy(idx_hbm, idx_all)
    pltpu.sync_copy(idx4_hbm.at[pl.ds(gbase, NGROUPS)], idx4_mine)

    def sig_step(c, _):
        pltpu.async_copy(
            sig_hbm.at[idx_all.at[pl.ds(base + c * SIG_CHUNK, SIG_CHUNK)]],
            sig_buf, sem_sig).wait()
        pltpu.sync_copy(sig_buf,
                        sigs_out.at[pl.ds(base + c * SIG_CHUNK, SIG_CHUNK)])
        return ()

    lax.fori_loop(0, B_PER_W // SIG_CHUNK, sig_step, ())

    rows = (row0, row1)
    outs = (out0, out1)
    sems_in = (sem_in0, sem_in1)
    sems_out = (sem_out0, sem_out1)

    def start_in(g, b):
        pltpu.async_copy(conn_hbm.at[idx4_mine.at[g]], rows[b], sems_in[b])

    def wait_in(b):
        pltpu.make_async_copy(conn_hbm.at[idx4_mine.at[0]], rows[b],
                              sems_in[b]).wait()

    def start_out(g, b):
        pltpu.async_copy(outs[b], conns_out.at[pl.ds(base + g * G, G)],
                         sems_out[b])

    def wait_out(b):
        pltpu.make_async_copy(outs[b], conns_out.at[pl.ds(base, G)],
                              sems_out[b]).wait()

    def compute(b):
        rbuf = rows[b]
        obuf = outs[b]

        @plsc.parallel_loop(0, B // LANES, 1, unroll=8)
        def col_step(k):
            cols = idx_all[pl.ds(k * LANES, LANES)]
            pos = k * LANES + lax.iota(jnp.int32, LANES)
            for j in range(G):
                rj = jnp.full((LANES,), j, jnp.int32)
                vals = plsc.load_gather(rbuf, [rj, cols])
                plsc.store_scatter(obuf, [rj, pos], vals)

    start_in(0, 0)
    start_in(1, 1)

    def group_pair(h, _):
        for b in range(2):
            g = 2 * h + b
            wait_in(b)

            @pl.when(h > 0)
            def _():
                wait_out(b)

            compute(b)
            start_out(g, b)

            @pl.when(g + 2 < NGROUPS)
            def _():
                start_in(g + 2, b)
        return ()

    lax.fori_loop(0, NGROUPS // 2, group_pair, ())
    wait_out(0)
    wait_out(1)


@jax.jit
def _run(indices, idx4, neuron_signatures, connection_strength):
    mesh = plsc.VectorSubcoreMesh(core_axis_name="c", subcore_axis_name="s",
                                  num_cores=NC, num_subcores=NS)
    kern = functools.partial(
        pl.kernel,
        out_type=[
            jax.ShapeDtypeStruct((B, D_STATE), jnp.float32),
            jax.ShapeDtypeStruct((B, B), jnp.float32),
        ],
        mesh=mesh,
        compiler_params=pltpu.CompilerParams(needs_layout_passes=False),
        scratch_types=[
            pltpu.VMEM((B,), jnp.int32),
            pltpu.VMEM((NGROUPS, G), jnp.int32),
            pltpu.VMEM((SIG_CHUNK, D_STATE), jnp.float32),
            pltpu.VMEM((G, N_NEURONS), jnp.float32),
            pltpu.VMEM((G, N_NEURONS), jnp.float32),
            pltpu.VMEM((G, B), jnp.float32),
            pltpu.VMEM((G, B), jnp.float32),
            pltpu.SemaphoreType.DMA,
            pltpu.SemaphoreType.DMA,
            pltpu.SemaphoreType.DMA,
            pltpu.SemaphoreType.DMA,
            pltpu.SemaphoreType.DMA,
        ],
    )(_body)
    return tuple(kern(indices, idx4, neuron_signatures, connection_strength))


def kernel(indices, neuron_signatures, connection_strength):
    idx = indices.astype(jnp.int32)
    return _run(idx, idx.reshape(B // G, G), neuron_signatures,
                connection_strength)


# speedup vs baseline: 1.6820x; 1.0288x over previous
import functools

import jax
import jax.numpy as jnp
from jax import lax
from jax.experimental import pallas as pl
from jax.experimental.pallas import tpu as pltpu
from jax.experimental.pallas import tpu_sc as plsc

N_NEURONS = 8192
D_STATE = 256
B = 4096

NC = 2
NS = 16
NW = NC * NS
B_PER_W = B // NW
LANES = 16
G = 4
NGROUPS = B_PER_W // G
SIG_CHUNK = 32


def _body(idx_hbm, idx4_hbm, sig_hbm, conn_hbm, sigs_out, conns_out,
          idx_all, idx4_mine, sig_buf,
          row0, row1, out0, out1,
          sem_in0, sem_in1, sem_out0, sem_out1, sem_sig):
    wid = lax.axis_index("s") * NC + lax.axis_index("c")
    base = wid * B_PER_W
    gbase = wid * NGROUPS

    pltpu.sync_copy(idx_hbm, idx_all)
    pltpu.sync_copy(idx4_hbm.at[pl.ds(gbase, NGROUPS)], idx4_mine)

    def sig_step(c, _):
        pltpu.async_copy(
            sig_hbm.at[idx_all.at[pl.ds(base + c * SIG_CHUNK, SIG_CHUNK)]],
            sig_buf, sem_sig).wait()
        pltpu.sync_copy(sig_buf,
                        sigs_out.at[pl.ds(base + c * SIG_CHUNK, SIG_CHUNK)])
        return ()

    rows = (row0, row1)
    outs = (out0, out1)
    sems_in = (sem_in0, sem_in1)
    sems_out = (sem_out0, sem_out1)

    def start_in(g, b):
        pltpu.async_copy(conn_hbm.at[idx4_mine.at[g]], rows[b], sems_in[b])

    def wait_in(b):
        pltpu.make_async_copy(conn_hbm.at[idx4_mine.at[0]], rows[b],
                              sems_in[b]).wait()

    def start_out(g, b):
        for j in range(G):
            pltpu.async_copy(outs[b].at[pl.ds(j * B, B)],
                             conns_out.at[base + g * G + j], sems_out[b])

    def wait_out(b):
        for j in range(G):
            pltpu.make_async_copy(outs[b].at[pl.ds(j * B, B)],
                                  conns_out.at[base + j], sems_out[b]).wait()

    def compute(b):
        rbuf = rows[b]
        obuf = outs[b]

        @plsc.parallel_loop(0, B // LANES, 1, unroll=8)
        def col_step(k):
            cols = idx_all[pl.ds(k * LANES, LANES)]
            for j in range(G):
                rj = jnp.full((LANES,), j, jnp.int32)
                vals = plsc.load_gather(rbuf, [rj, cols])
                obuf[pl.ds(j * B + k * LANES, LANES)] = vals

    start_in(0, 0)
    start_in(1, 1)
    lax.fori_loop(0, B_PER_W // SIG_CHUNK, sig_step, ())

    def group_pair(h, _):
        for b in range(2):
            g = 2 * h + b
            wait_in(b)

            @pl.when(h > 0)
            def _():
                wait_out(b)

            compute(b)
            start_out(g, b)

            @pl.when(g + 2 < NGROUPS)
            def _():
                start_in(g + 2, b)
        return ()

    lax.fori_loop(0, NGROUPS // 2, group_pair, ())
    wait_out(0)
    wait_out(1)


@jax.jit
def _run(indices, idx4, neuron_signatures, connection_strength):
    mesh = plsc.VectorSubcoreMesh(core_axis_name="c", subcore_axis_name="s",
                                  num_cores=NC, num_subcores=NS)
    kern = functools.partial(
        pl.kernel,
        out_type=[
            jax.ShapeDtypeStruct((B, D_STATE), jnp.float32),
            jax.ShapeDtypeStruct((B, B), jnp.float32),
        ],
        mesh=mesh,
        compiler_params=pltpu.CompilerParams(needs_layout_passes=False),
        scratch_types=[
            pltpu.VMEM((B,), jnp.int32),
            pltpu.VMEM((NGROUPS, G), jnp.int32),
            pltpu.VMEM((SIG_CHUNK, D_STATE), jnp.float32),
            pltpu.VMEM((G, N_NEURONS), jnp.float32),
            pltpu.VMEM((G, N_NEURONS), jnp.float32),
            pltpu.VMEM((G * B,), jnp.float32),
            pltpu.VMEM((G * B,), jnp.float32),
            pltpu.SemaphoreType.DMA,
            pltpu.SemaphoreType.DMA,
            pltpu.SemaphoreType.DMA,
            pltpu.SemaphoreType.DMA,
            pltpu.SemaphoreType.DMA,
        ],
    )(_body)
    return tuple(kern(indices, idx4, neuron_signatures, connection_strength))


def kernel(indices, neuron_signatures, connection_strength):
    idx = indices.astype(jnp.int32)
    return _run(idx, idx.reshape(B // G, G), neuron_signatures,
                connection_strength)


# speedup vs baseline: 1.7045x; 1.0134x over previous
import functools

import jax
import jax.numpy as jnp
from jax import lax
from jax.experimental import pallas as pl
from jax.experimental.pallas import tpu as pltpu
from jax.experimental.pallas import tpu_sc as plsc

N_NEURONS = 8192
D_STATE = 256
B = 4096

NC = 2
NS = 16
NW = NC * NS
B_PER_W = B // NW
LANES = 16
G = 4
NGROUPS = B_PER_W // G
SIG_CHUNK = 32


def _body(idx_hbm, idx4_hbm, sig_hbm, conn_hbm, sigs_out, conns_out,
          idx_all, idx4_mine, sig_buf,
          row0, row1, out0, out1, shared_out,
          sem_in0, sem_in1, sem_out0, sem_out1, sem_sig):
    wid = lax.axis_index("s") * NC + lax.axis_index("c")
    base = wid * B_PER_W
    gbase = wid * NGROUPS

    pltpu.sync_copy(idx_hbm, idx_all)
    pltpu.sync_copy(idx4_hbm.at[pl.ds(gbase, NGROUPS)], idx4_mine)

    def sig_step(c, _):
        pltpu.async_copy(
            sig_hbm.at[idx_all.at[pl.ds(base + c * SIG_CHUNK, SIG_CHUNK)]],
            sig_buf, sem_sig).wait()
        pltpu.sync_copy(sig_buf,
                        sigs_out.at[pl.ds(base + c * SIG_CHUNK, SIG_CHUNK)])
        return ()

    rows = (row0, row1)
    outs = (out0, out1)
    sems_in = (sem_in0, sem_in1)
    sems_out = (sem_out0, sem_out1)

    def start_in(g, b):
        pltpu.async_copy(conn_hbm.at[idx4_mine.at[g]], rows[b], sems_in[b])

    def wait_in(b):
        pltpu.make_async_copy(conn_hbm.at[idx4_mine.at[0]], rows[b],
                              sems_in[b]).wait()

    sid = lax.axis_index("s")

    def start_out(g, b):
        pltpu.sync_copy(outs[b], shared_out.at[sid])
        for j in range(G):
            pltpu.async_copy(shared_out.at[sid].at[pl.ds(j * B, B)],
                             conns_out.at[base + g * G + j], sems_out[0])

    def wait_out(b):
        for j in range(G):
            pltpu.make_async_copy(shared_out.at[sid].at[pl.ds(j * B, B)],
                                  conns_out.at[base + j], sems_out[0]).wait()

    def compute(b):
        rbuf = rows[b]
        obuf = outs[b]

        @plsc.parallel_loop(0, B // LANES, 1, unroll=8)
        def col_step(k):
            cols = idx_all[pl.ds(k * LANES, LANES)]
            for j in range(G):
                rj = jnp.full((LANES,), j, jnp.int32)
                vals = plsc.load_gather(rbuf, [rj, cols])
                obuf[pl.ds(j * B + k * LANES, LANES)] = vals

    start_in(0, 0)
    start_in(1, 1)
    lax.fori_loop(0, B_PER_W // SIG_CHUNK, sig_step, ())

    def group_pair(h, _):
        for b in range(2):
            g = 2 * h + b
            wait_in(b)
            compute(b)

            @pl.when(g > 0)
            def _():
                wait_out(b)

            start_out(g, b)

            @pl.when(g + 2 < NGROUPS)
            def _():
                start_in(g + 2, b)
        return ()

    lax.fori_loop(0, NGROUPS // 2, group_pair, ())
    wait_out(0)


@jax.jit
def _run(indices, idx4, neuron_signatures, connection_strength):
    mesh = plsc.VectorSubcoreMesh(core_axis_name="c", subcore_axis_name="s",
                                  num_cores=NC, num_subcores=NS)
    kern = functools.partial(
        pl.kernel,
        out_type=[
            jax.ShapeDtypeStruct((B, D_STATE), jnp.float32),
            jax.ShapeDtypeStruct((B, B), jnp.float32),
        ],
        mesh=mesh,
        compiler_params=pltpu.CompilerParams(needs_layout_passes=False),
        scratch_types=[
            pltpu.VMEM((B,), jnp.int32),
            pltpu.VMEM((NGROUPS, G), jnp.int32),
            pltpu.VMEM((SIG_CHUNK, D_STATE), jnp.float32),
            pltpu.VMEM((G, N_NEURONS), jnp.float32),
            pltpu.VMEM((G, N_NEURONS), jnp.float32),
            pltpu.VMEM((G * B,), jnp.float32),
            pltpu.VMEM((G * B,), jnp.float32),
            pltpu.VMEM_SHARED((NS, G * B), jnp.float32),
            pltpu.SemaphoreType.DMA,
            pltpu.SemaphoreType.DMA,
            pltpu.SemaphoreType.DMA,
            pltpu.SemaphoreType.DMA,
            pltpu.SemaphoreType.DMA,
        ],
    )(_body)
    return tuple(kern(indices, idx4, neuron_signatures, connection_strength))


def kernel(indices, neuron_signatures, connection_strength):
    idx = indices.astype(jnp.int32)
    return _run(idx, idx.reshape(B // G, G), neuron_signatures,
                connection_strength)
